# baseline (speedup 1.0000x reference)
import jax
import jax.numpy as jnp
from jax.experimental import pallas as pl
from jax.experimental.pallas import tpu as pltpu

_BM = 1024
_NT = 4
_TR = 1024


def _readout_matmul_kernel(w_hbm, a_ref, o_ref, wbf_ref, stage_ref, sems):
    i = pl.program_id(0)

    def _copy(t, slot):
        return pltpu.make_async_copy(
            w_hbm.at[pl.ds(t * _TR, _TR), :],
            stage_ref.at[slot],
            sems.at[t],
        )

    a = a_ref[...].astype(jnp.bfloat16)

    @pl.when(i == 0)
    def _stream_first_row():
        _copy(0, 0).start()
        _copy(1, 1).start()
        for t in range(_NT):
            slot = t % 2
            _copy(t, slot).wait()
            wbf_ref[pl.ds(t * _TR, _TR), :] = stage_ref[slot].astype(jnp.bfloat16)
            if t + 2 < _NT:
                _copy(t + 2, slot).start()
            o_ref[:, pl.ds(t * _TR, _TR)] = jax.lax.dot_general(
                a, wbf_ref[pl.ds(t * _TR, _TR), :],
                dimension_numbers=(((1,), (1,)), ((), ())),
                preferred_element_type=jnp.float32,
            )

    @pl.when(i > 0)
    def _steady():
        o_ref[...] = jax.lax.dot_general(
            a, wbf_ref[...],
            dimension_numbers=(((1,), (1,)), ((), ())),
            preferred_element_type=jnp.float32,
        )


def kernel(embed, emb_weight):
    m, d = embed.shape
    l, _ = emb_weight.shape
    return pl.pallas_call(
        _readout_matmul_kernel,
        grid=(m // _BM,),
        in_specs=[
            pl.BlockSpec(memory_space=pltpu.MemorySpace.HBM),
            pl.BlockSpec((_BM, d), lambda i: (i, 0)),
        ],
        out_specs=pl.BlockSpec((_BM, l), lambda i: (i, 0)),
        out_shape=jax.ShapeDtypeStruct((m, l), jnp.float32),
        scratch_shapes=[
            pltpu.VMEM((l, d), jnp.bfloat16),
            pltpu.VMEM((2, _TR, d), jnp.float32),
            pltpu.SemaphoreType.DMA((_NT,)),
        ],
    )(emb_weight, embed)
